# baseline (speedup 1.0000x reference)
import functools

import jax
import jax.numpy as jnp
from jax import lax
from jax.experimental import pallas as pl
from jax.experimental.pallas import tpu as pltpu
from jax.experimental.pallas import tpu_sc as plsc

NC = 2
NS = 16
NW = NC * NS
L = 16
CH = 128
G = 16


def _make_sc_agg(n_nodes, w0, w1, d, with_deg):
    zrows = 8 * (-(-n_nodes // (8 * NS)))
    n_rows = zrows * NS
    outw = 8 * (n_nodes // (8 * NS))
    outw_last = n_nodes - outw * (NS - 1)
    mesh = plsc.VectorSubcoreMesh(core_axis_name="c", subcore_axis_name="s")

    out_type = [jax.ShapeDtypeStruct((NC, n_nodes, d), jnp.float32)]
    scratch = [
        pltpu.VMEM((G, CH), jnp.int32),
        pltpu.VMEM((G, CH), jnp.int32),
        pltpu.VMEM((CH, d), jnp.float32),
        pltpu.VMEM((CH, d), jnp.float32),
        pltpu.VMEM_SHARED((n_rows, d), jnp.float32),
        pltpu.SemaphoreType.DMA,
        pltpu.SemaphoreType.DMA,
        pltpu.SemaphoreType.DMA,
        pltpu.SemaphoreType.DMA,
    ]
    if with_deg:
        out_type.append(jax.ShapeDtypeStruct((NC, NS, zrows), jnp.float32))
        scratch += [
            pltpu.VMEM((CH,), jnp.float32),
            pltpu.VMEM((CH,), jnp.float32),
            pltpu.VMEM_SHARED((n_rows,), jnp.float32),
            pltpu.SemaphoreType.DMA,
        ]

    def body(x_hbm, src_hbm, dst_hbm, *rest):
        if with_deg:
            (agg_hbm, deg_hbm, src_v, dst_v, rows0, rows1, accum,
             gsem0, gsem1, ssem0, ssem1,
             ones_v, zdeg_v, degacc, dsem) = rest
        else:
            (agg_hbm, src_v, dst_v, rows0, rows1, accum,
             gsem0, gsem1, ssem0, ssem1) = rest
        rows_v = rows0
        rbuf = (rows0, rows1)
        gsem = (gsem0, gsem1)
        ssem = (ssem0, ssem1)
        cid = lax.axis_index("c")
        sid = lax.axis_index("s")

        @pl.loop(0, CH)
        def _(i):
            for j in range(d // L):
                rows_v[i, pl.ds(j * L, L)] = jnp.zeros((L,), jnp.float32)

        if with_deg:
            for j in range(CH // L):
                ones_v[pl.ds(j * L, L)] = jnp.ones((L,), jnp.float32)
                zdeg_v[pl.ds(j * L, L)] = jnp.zeros((L,), jnp.float32)

        zbase = pl.multiple_of(sid * zrows, 8)
        nfull = zrows // CH
        for k in range(nfull):
            pltpu.sync_copy(rows_v, accum.at[pl.ds(zbase + k * CH, CH)])
        rem = zrows - nfull * CH
        if rem:
            pltpu.sync_copy(rows_v.at[pl.ds(0, rem)],
                            accum.at[pl.ds(zbase + nfull * CH, rem)])
        if with_deg:
            for k in range(nfull):
                pltpu.sync_copy(zdeg_v, degacc.at[pl.ds(zbase + k * CH, CH)])
            if rem:
                pltpu.sync_copy(zdeg_v.at[pl.ds(0, rem)],
                                degacc.at[pl.ds(zbase + nfull * CH, rem)])

        plsc.subcore_barrier()

        def run_chunks(nchunks, start):
            @pl.loop(0, nchunks // G)
            def _(g):
                gbase = pl.multiple_of(start + g * G, 8)
                pltpu.sync_copy(src_hbm.at[pl.ds(gbase, G)], src_v)
                pltpu.sync_copy(dst_hbm.at[pl.ds(gbase, G)], dst_v)

                gd = [None] * G
                sd = [None] * G
                dd = []
                for i in range(G):
                    b = i & 1
                    if i >= 2:
                        sd[i - 2].wait()
                    gd[i] = pltpu.async_copy(x_hbm.at[src_v.at[i]], rbuf[b],
                                             gsem[b])
                    if i >= 1:
                        gd[i - 1].wait()
                        sd[i - 1] = pltpu.async_copy(
                            rbuf[1 - b], accum.at[dst_v.at[i - 1]], ssem[1 - b],
                            add=True)
                        if with_deg:
                            dd.append(pltpu.async_copy(
                                ones_v, degacc.at[dst_v.at[i - 1]], dsem,
                                add=True))
                bl = (G - 1) & 1
                gd[G - 1].wait()
                sd[G - 1] = pltpu.async_copy(
                    rbuf[bl], accum.at[dst_v.at[G - 1]], ssem[bl], add=True)
                if with_deg:
                    dd.append(pltpu.async_copy(
                        ones_v, degacc.at[dst_v.at[G - 1]], dsem, add=True))
                sd[G - 2].wait()
                sd[G - 1].wait()
                for dsc in dd:
                    dsc.wait()

        if w0:
            @pl.when(cid == 0)
            def _():
                run_chunks(w0, pl.multiple_of(sid * w0, 8))
        if w1:
            @pl.when(cid == 1)
            def _():
                run_chunks(w1, pl.multiple_of(NS * w0 + sid * w1, 8))

        plsc.subcore_barrier()

        obase = pl.multiple_of(sid * outw, 8)

        @pl.when(sid < NS - 1)
        def _():
            pltpu.sync_copy(accum.at[pl.ds(obase, outw)],
                            agg_hbm.at[cid, pl.ds(obase, outw)])

        @pl.when(sid == NS - 1)
        def _():
            base = (NS - 1) * outw
            pltpu.sync_copy(accum.at[pl.ds(base, outw_last)],
                            agg_hbm.at[cid, pl.ds(base, outw_last)])

        if with_deg:
            pltpu.sync_copy(degacc.at[pl.ds(zbase, zrows)],
                            deg_hbm.at[cid, sid])

    return pl.kernel(
        body, out_type=out_type, mesh=mesh, scratch_types=scratch,
        compiler_params=pltpu.CompilerParams(use_tc_tiling_on_sc=False))


def _dense_body(aggp, degp, x, wl, b, wr, o):
    agg = aggp[0] + aggp[1]
    mean = agg * (1.0 / jnp.maximum(degp[...], 1.0))
    h = (jnp.dot(mean, wl[...], preferred_element_type=jnp.float32)
         + b[...]
         + jnp.dot(x[...], wr[...], preferred_element_type=jnp.float32))
    o[...] = jnp.maximum(h, 0.0)


def _dense_head_body(aggp, degp, x, wl, b, wr, wc, bc, o):
    agg = aggp[0] + aggp[1]
    mean = agg * (1.0 / jnp.maximum(degp[...], 1.0))
    h = (jnp.dot(mean, wl[...], preferred_element_type=jnp.float32)
         + b[...]
         + jnp.dot(x[...], wr[...], preferred_element_type=jnp.float32))
    h = jnp.maximum(h, 0.0)
    o[...] = jnp.dot(h, wc[...], preferred_element_type=jnp.float32) + bc[...]


def _dense(aggp, degp, x, wl, b, wr, br):
    n, d = x.shape
    grid = (n // br,)
    return pl.pallas_call(
        _dense_body,
        grid=grid,
        in_specs=[
            pl.BlockSpec((NC, br, d), lambda i: (0, i, 0)),
            pl.BlockSpec((br, 1), lambda i: (i, 0)),
            pl.BlockSpec((br, d), lambda i: (i, 0)),
            pl.BlockSpec((d, d), lambda i: (0, 0)),
            pl.BlockSpec((1, d), lambda i: (0, 0)),
            pl.BlockSpec((d, d), lambda i: (0, 0)),
        ],
        out_specs=pl.BlockSpec((br, d), lambda i: (i, 0)),
        out_shape=jax.ShapeDtypeStruct((n, d), jnp.float32),
    )(aggp, degp, x, wl, b, wr)


def _dense_head(aggp, degp, x, wl, b, wr, wc, bc, br):
    n, d = x.shape
    c = wc.shape[1]
    grid = (n // br,)
    return pl.pallas_call(
        _dense_head_body,
        grid=grid,
        in_specs=[
            pl.BlockSpec((NC, br, d), lambda i: (0, i, 0)),
            pl.BlockSpec((br, 1), lambda i: (i, 0)),
            pl.BlockSpec((br, d), lambda i: (i, 0)),
            pl.BlockSpec((d, d), lambda i: (0, 0)),
            pl.BlockSpec((1, d), lambda i: (0, 0)),
            pl.BlockSpec((d, d), lambda i: (0, 0)),
            pl.BlockSpec((d, c), lambda i: (0, 0)),
            pl.BlockSpec((1, c), lambda i: (0, 0)),
        ],
        out_specs=pl.BlockSpec((br, c), lambda i: (i, 0)),
        out_shape=jax.ShapeDtypeStruct((n, c), jnp.float32),
    )(aggp, degp, x, wl, b, wr, wc, bc)


def kernel(x, edge_index, W1_l, b1_l, W1_r, W2_l, b2_l, W2_r, W_c, b_c):
    n, d = x.shape
    e = edge_index.shape[1]
    nch = G * (-(-e // (NW * CH * G)))
    w0 = G * (((2 * nch) // 5) // G)
    w1 = 2 * nch - w0
    pad = NS * (w0 + w1) * CH - e

    src = jnp.concatenate(
        [edge_index[0], jnp.zeros((pad,), jnp.int32)]).reshape(-1, CH)
    dst = jnp.concatenate(
        [edge_index[1], jnp.full((pad,), n, jnp.int32)]).reshape(-1, CH)

    agg_deg = _make_sc_agg(n, w0, w1, d, with_deg=True)
    agg_only = _make_sc_agg(n, w0, w1, d, with_deg=False)

    b1 = b1_l.reshape(1, -1)
    b2 = b2_l.reshape(1, -1)
    bc = b_c.reshape(1, -1)

    agg1p, degp = agg_deg(x, src, dst)
    deg = (degp[0] + degp[1]).reshape(-1)[:n].reshape(n, 1)
    h = _dense(agg1p, deg, x, W1_l.T, b1, W1_r.T, br=1000)
    agg2p = agg_only(h, src, dst)
    if isinstance(agg2p, (list, tuple)):
        agg2p = agg2p[0]
    out = _dense_head(agg2p, deg, h, W2_l.T, b2, W2_r.T, W_c.T, bc, br=1000)
    return out


# speedup vs baseline: 1.1697x; 1.1697x over previous
import functools

import jax
import jax.numpy as jnp
from jax import lax
from jax.experimental import pallas as pl
from jax.experimental.pallas import tpu as pltpu
from jax.experimental.pallas import tpu_sc as plsc

NC = 2
NS = 16
NW = NC * NS
L = 16
CH = 128
G = 16


def _make_sc_agg(n_nodes, w0, w1, d, with_deg):
    zrows = 8 * (-(-n_nodes // (8 * NS)))
    n_rows = zrows * NS
    outw = 8 * (n_nodes // (8 * NS))
    outw_last = n_nodes - outw * (NS - 1)
    mesh = plsc.VectorSubcoreMesh(core_axis_name="c", subcore_axis_name="s")

    out_type = [jax.ShapeDtypeStruct((NC, n_nodes, d), jnp.float32)]
    scratch = [
        pltpu.VMEM((G, CH), jnp.int32),
        pltpu.VMEM((G, CH), jnp.int32),
        pltpu.VMEM((CH, d), jnp.float32),
        pltpu.VMEM((CH, d), jnp.float32),
        pltpu.VMEM_SHARED((n_rows, d), jnp.float32),
        pltpu.SemaphoreType.DMA,
        pltpu.SemaphoreType.DMA,
        pltpu.SemaphoreType.DMA,
        pltpu.SemaphoreType.DMA,
    ]
    if with_deg:
        out_type.append(jax.ShapeDtypeStruct((NC, NS, zrows), jnp.float32))
        scratch += [
            pltpu.VMEM((CH,), jnp.float32),
            pltpu.VMEM((CH,), jnp.float32),
            pltpu.VMEM_SHARED((n_rows,), jnp.float32),
            pltpu.SemaphoreType.DMA,
        ]

    def body(x_hbm, src_hbm, dst_hbm, *rest):
        if with_deg:
            (agg_hbm, deg_hbm, src_v, dst_v, rows0, rows1, accum,
             gsem0, gsem1, ssem0, ssem1,
             ones_v, zdeg_v, degacc, dsem) = rest
        else:
            (agg_hbm, src_v, dst_v, rows0, rows1, accum,
             gsem0, gsem1, ssem0, ssem1) = rest
        rows_v = rows0
        rbuf = (rows0, rows1)
        gsem = (gsem0, gsem1)
        ssem = (ssem0, ssem1)
        cid = lax.axis_index("c")
        sid = lax.axis_index("s")

        @pl.loop(0, CH)
        def _(i):
            for j in range(d // L):
                rows_v[i, pl.ds(j * L, L)] = jnp.zeros((L,), jnp.float32)

        if with_deg:
            for j in range(CH // L):
                ones_v[pl.ds(j * L, L)] = jnp.ones((L,), jnp.float32)
                zdeg_v[pl.ds(j * L, L)] = jnp.zeros((L,), jnp.float32)

        zbase = pl.multiple_of(sid * zrows, 8)
        nfull = zrows // CH
        for k in range(nfull):
            pltpu.sync_copy(rows_v, accum.at[pl.ds(zbase + k * CH, CH)])
        rem = zrows - nfull * CH
        if rem:
            pltpu.sync_copy(rows_v.at[pl.ds(0, rem)],
                            accum.at[pl.ds(zbase + nfull * CH, rem)])
        if with_deg:
            for k in range(nfull):
                pltpu.sync_copy(zdeg_v, degacc.at[pl.ds(zbase + k * CH, CH)])
            if rem:
                pltpu.sync_copy(zdeg_v.at[pl.ds(0, rem)],
                                degacc.at[pl.ds(zbase + nfull * CH, rem)])

        plsc.subcore_barrier()

        def run_chunks(nchunks, start):
            @pl.loop(0, nchunks // G)
            def _(g):
                gbase = pl.multiple_of(start + g * G, 8)
                pltpu.sync_copy(src_hbm.at[pl.ds(gbase, G)], src_v)
                pltpu.sync_copy(dst_hbm.at[pl.ds(gbase, G)], dst_v)

                gd = [None] * G
                sd = [None] * G
                dd = []
                for i in range(G):
                    b = i & 1
                    if i >= 2:
                        sd[i - 2].wait()
                    gd[i] = pltpu.async_copy(x_hbm.at[src_v.at[i]], rbuf[b],
                                             gsem[b])
                    if i >= 1:
                        gd[i - 1].wait()
                        sd[i - 1] = pltpu.async_copy(
                            rbuf[1 - b], accum.at[dst_v.at[i - 1]], ssem[1 - b],
                            add=True)
                        if with_deg:
                            dd.append(pltpu.async_copy(
                                ones_v, degacc.at[dst_v.at[i - 1]], dsem,
                                add=True))
                bl = (G - 1) & 1
                gd[G - 1].wait()
                sd[G - 1] = pltpu.async_copy(
                    rbuf[bl], accum.at[dst_v.at[G - 1]], ssem[bl], add=True)
                if with_deg:
                    dd.append(pltpu.async_copy(
                        ones_v, degacc.at[dst_v.at[G - 1]], dsem, add=True))
                sd[G - 2].wait()
                sd[G - 1].wait()
                for dsc in dd:
                    dsc.wait()

        if w0:
            @pl.when(cid == 0)
            def _():
                run_chunks(w0, pl.multiple_of(sid * w0, 8))
        if w1:
            @pl.when(cid == 1)
            def _():
                run_chunks(w1, pl.multiple_of(NS * w0 + sid * w1, 8))

        plsc.subcore_barrier()

        obase = pl.multiple_of(sid * outw, 8)

        @pl.when(sid < NS - 1)
        def _():
            pltpu.sync_copy(accum.at[pl.ds(obase, outw)],
                            agg_hbm.at[cid, pl.ds(obase, outw)])

        @pl.when(sid == NS - 1)
        def _():
            base = (NS - 1) * outw
            pltpu.sync_copy(accum.at[pl.ds(base, outw_last)],
                            agg_hbm.at[cid, pl.ds(base, outw_last)])

        if with_deg:
            pltpu.sync_copy(degacc.at[pl.ds(zbase, zrows)],
                            deg_hbm.at[cid, sid])

    return pl.kernel(
        body, out_type=out_type, mesh=mesh, scratch_types=scratch,
        compiler_params=pltpu.CompilerParams(use_tc_tiling_on_sc=False))


def _dense_body(aggp, degp, x, wl, b, wr, o):
    agg = aggp[0] + aggp[1]
    mean = agg * (1.0 / jnp.maximum(degp[...], 1.0))
    h = (jnp.dot(mean, wl[...], preferred_element_type=jnp.float32)
         + b[...]
         + jnp.dot(x[...], wr[...], preferred_element_type=jnp.float32))
    o[...] = jnp.maximum(h, 0.0)


def _dense_head_body(aggp, degp, x, wl, b, wr, wc, bc, o):
    agg = aggp[0] + aggp[1]
    mean = agg * (1.0 / jnp.maximum(degp[...], 1.0))
    h = (jnp.dot(mean, wl[...], preferred_element_type=jnp.float32)
         + b[...]
         + jnp.dot(x[...], wr[...], preferred_element_type=jnp.float32))
    h = jnp.maximum(h, 0.0)
    o[...] = jnp.dot(h, wc[...], preferred_element_type=jnp.float32) + bc[...]


def _dense(aggp, degp, x, wl, b, wr, br):
    n, d = x.shape
    grid = (n // br,)
    return pl.pallas_call(
        _dense_body,
        grid=grid,
        in_specs=[
            pl.BlockSpec((NC, br, d), lambda i: (0, i, 0)),
            pl.BlockSpec((br, 1), lambda i: (i, 0)),
            pl.BlockSpec((br, d), lambda i: (i, 0)),
            pl.BlockSpec((d, d), lambda i: (0, 0)),
            pl.BlockSpec((1, d), lambda i: (0, 0)),
            pl.BlockSpec((d, d), lambda i: (0, 0)),
        ],
        out_specs=pl.BlockSpec((br, d), lambda i: (i, 0)),
        out_shape=jax.ShapeDtypeStruct((n, d), jnp.float32),
    )(aggp, degp, x, wl, b, wr)


def _dense_head(aggp, degp, x, wl, b, wr, wc, bc, br):
    n, d = x.shape
    c = wc.shape[1]
    grid = (n // br,)
    return pl.pallas_call(
        _dense_head_body,
        grid=grid,
        in_specs=[
            pl.BlockSpec((NC, br, d), lambda i: (0, i, 0)),
            pl.BlockSpec((br, 1), lambda i: (i, 0)),
            pl.BlockSpec((br, d), lambda i: (i, 0)),
            pl.BlockSpec((d, d), lambda i: (0, 0)),
            pl.BlockSpec((1, d), lambda i: (0, 0)),
            pl.BlockSpec((d, d), lambda i: (0, 0)),
            pl.BlockSpec((d, c), lambda i: (0, 0)),
            pl.BlockSpec((1, c), lambda i: (0, 0)),
        ],
        out_specs=pl.BlockSpec((br, c), lambda i: (i, 0)),
        out_shape=jax.ShapeDtypeStruct((n, c), jnp.float32),
    )(aggp, degp, x, wl, b, wr, wc, bc)


def kernel(x, edge_index, W1_l, b1_l, W1_r, W2_l, b2_l, W2_r, W_c, b_c):
    n, d = x.shape
    e = edge_index.shape[1]
    nch = G * (-(-e // (NW * CH * G)))
    w1 = G * (((2 * nch) // 5) // G)
    w0 = 2 * nch - w1
    pad = NS * (w0 + w1) * CH - e

    src = jnp.concatenate(
        [edge_index[0], jnp.zeros((pad,), jnp.int32)]).reshape(-1, CH)
    dst = jnp.concatenate(
        [edge_index[1], jnp.full((pad,), n, jnp.int32)]).reshape(-1, CH)

    agg_deg = _make_sc_agg(n, w0, w1, d, with_deg=True)
    agg_only = _make_sc_agg(n, w0, w1, d, with_deg=False)

    b1 = b1_l.reshape(1, -1)
    b2 = b2_l.reshape(1, -1)
    bc = b_c.reshape(1, -1)

    agg1p, degp = agg_deg(x, src, dst)
    deg = (degp[0] + degp[1]).reshape(-1)[:n].reshape(n, 1)
    h = _dense(agg1p, deg, x, W1_l.T, b1, W1_r.T, br=1000)
    agg2p = agg_only(h, src, dst)
    if isinstance(agg2p, (list, tuple)):
        agg2p = agg2p[0]
    out = _dense_head(agg2p, deg, h, W2_l.T, b2, W2_r.T, W_c.T, bc, br=1000)
    return out
